# Initial kernel scaffold; baseline (speedup 1.0000x reference)
import jax
import jax.numpy as jnp
from jax.experimental import pallas as pl


def kernel(seq, adj, W, b, a):
    raise NotImplementedError("write your pallas kernel here")


# speedup vs baseline: 1.0252x; 1.0252x over previous
import jax
import jax.numpy as jnp
from jax.experimental import pallas as pl
from jax.experimental.pallas import tpu as pltpu

_N = 10000
_FT = 128
_BM = 200


def _gcn_block_kernel(seq_ref, w_ref, b_ref, a_ref, adj_ref, out_ref, h_ref):
    i = pl.program_id(0)

    @pl.when(i == 0)
    def _compute_h():
        s = seq_ref[...].astype(jnp.bfloat16)
        w = w_ref[...].astype(jnp.bfloat16)
        h = jax.lax.dot_general(
            s, w, (((1,), (1,)), ((), ())),
            preferred_element_type=jnp.float32,
        ) + b_ref[...]
        h_ref[...] = h.astype(jnp.bfloat16)

    adj_b = adj_ref[...].astype(jnp.bfloat16)
    o = jnp.dot(adj_b, h_ref[...], preferred_element_type=jnp.float32)
    alpha = a_ref[0, 0]
    out_ref[...] = jnp.where(o >= 0, o, alpha * o)


def kernel(seq, adj, W, b, a):
    seq2 = seq.reshape(_N, _FT)
    adj2 = adj.reshape(_N, _N)
    b2 = b.reshape(1, _FT)
    a2 = a.reshape(1, 1)

    out = pl.pallas_call(
        _gcn_block_kernel,
        grid=(_N // _BM,),
        in_specs=[
            pl.BlockSpec((_N, _FT), lambda i: (0, 0)),
            pl.BlockSpec((_FT, _FT), lambda i: (0, 0)),
            pl.BlockSpec((1, _FT), lambda i: (0, 0)),
            pl.BlockSpec((1, 1), lambda i: (0, 0)),
            pl.BlockSpec((_BM, _N), lambda i: (i, 0)),
        ],
        out_specs=pl.BlockSpec((_BM, _FT), lambda i: (i, 0)),
        out_shape=jax.ShapeDtypeStruct((_N, _FT), jnp.float32),
        scratch_shapes=[pltpu.VMEM((_N, _FT), jnp.bfloat16)],
    )(seq2, W, b2, a2, adj2)
    return out.reshape(1, _N, _FT)


# speedup vs baseline: 1.0399x; 1.0143x over previous
import jax
import jax.numpy as jnp
from jax.experimental import pallas as pl
from jax.experimental.pallas import tpu as pltpu

_N = 10000
_FT = 128
_BM = 400


def _gcn_block_kernel(seq_ref, w_ref, b_ref, a_ref, adj_ref, out_ref, h_ref):
    i = pl.program_id(0)

    @pl.when(i == 0)
    def _compute_h():
        s = seq_ref[...].astype(jnp.bfloat16)
        w = w_ref[...].astype(jnp.bfloat16)
        h = jax.lax.dot_general(
            s, w, (((1,), (1,)), ((), ())),
            preferred_element_type=jnp.float32,
        ) + b_ref[...]
        h_ref[...] = h.astype(jnp.bfloat16)

    adj_b = adj_ref[...].astype(jnp.bfloat16)
    o = jnp.dot(adj_b, h_ref[...], preferred_element_type=jnp.float32)
    alpha = a_ref[0, 0]
    out_ref[...] = jnp.where(o >= 0, o, alpha * o)


def kernel(seq, adj, W, b, a):
    seq2 = seq.reshape(_N, _FT)
    adj2 = adj.reshape(_N, _N)
    b2 = b.reshape(1, _FT)
    a2 = a.reshape(1, 1)

    out = pl.pallas_call(
        _gcn_block_kernel,
        grid=(_N // _BM,),
        in_specs=[
            pl.BlockSpec((_N, _FT), lambda i: (0, 0)),
            pl.BlockSpec((_FT, _FT), lambda i: (0, 0)),
            pl.BlockSpec((1, _FT), lambda i: (0, 0)),
            pl.BlockSpec((1, 1), lambda i: (0, 0)),
            pl.BlockSpec((_BM, _N), lambda i: (i, 0)),
        ],
        out_specs=pl.BlockSpec((_BM, _FT), lambda i: (i, 0)),
        out_shape=jax.ShapeDtypeStruct((_N, _FT), jnp.float32),
        scratch_shapes=[pltpu.VMEM((_N, _FT), jnp.bfloat16)],
    )(seq2, W, b2, a2, adj2)
    return out.reshape(1, _N, _FT)
